# Initial kernel scaffold; baseline (speedup 1.0000x reference)
import functools

import jax
import jax.numpy as jnp
from jax import lax
from jax.experimental import pallas as pl
from jax.experimental.pallas import tpu as pltpu
from jax.experimental.pallas import tpu_sc as plsc

N = 10000
E = 320000
C = 128
NC = 2
NS = 16
NW = NC * NS
EPW = E // NW
K = 80
NCH = EPW // K
RPT = N // NS
RCH = 125


def _vsc_mesh():
    return plsc.VectorSubcoreMesh(core_axis_name="c", subcore_axis_name="s")


@functools.partial(
    pl.kernel,
    out_type=jax.ShapeDtypeStruct((NC, 2, N), jnp.float32),
    mesh=_vsc_mesh(),
    scratch_types=[
        pltpu.VMEM((NCH, K), jnp.int32),
        pltpu.VMEM((NCH, K), jnp.int32),
        pltpu.VMEM((K,), jnp.float32),
        pltpu.VMEM((2000,), jnp.float32),
        pltpu.VMEM_SHARED((N,), jnp.float32),
        pltpu.VMEM_SHARED((N,), jnp.float32),
    ],
)
def _deg(src_hbm, dst_hbm, degp_hbm, sidx, didx, ones, zbuf, acc_s, acc_d):
    cid = lax.axis_index("c")
    sid = lax.axis_index("s")
    wid = cid * NS + sid

    @pl.when(sid == 0)
    def _init():
        def zrow(i, carry):
            zbuf[pl.ds(i * 16, 16)] = jnp.zeros((16,), jnp.float32)
            return carry
        lax.fori_loop(0, 2000 // 16, zrow, None)
        for t in range(N // 2000):
            pltpu.sync_copy(zbuf, acc_s.at[pl.ds(t * 2000, 2000)])
            pltpu.sync_copy(zbuf, acc_d.at[pl.ds(t * 2000, 2000)])

    for c16 in range(K // 16):
        ones[pl.ds(c16 * 16, 16)] = jnp.ones((16,), jnp.float32)

    plsc.subcore_barrier()

    pltpu.sync_copy(src_hbm.at[wid], sidx)
    pltpu.sync_copy(dst_hbm.at[wid], didx)

    def chunk(j, carry):
        pltpu.sync_copy(ones, acc_s.at[sidx.at[j]], add=True)
        pltpu.sync_copy(ones, acc_d.at[didx.at[j]], add=True)
        return carry
    lax.fori_loop(0, NCH, chunk, None)

    plsc.subcore_barrier()

    @pl.when(sid == 0)
    def _writeout():
        pltpu.sync_copy(acc_s, degp_hbm.at[cid, 0])
        pltpu.sync_copy(acc_d, degp_hbm.at[cid, 1])


@functools.partial(
    pl.kernel,
    out_type=jax.ShapeDtypeStruct((NC, N, C), jnp.float32),
    mesh=_vsc_mesh(),
    scratch_types=[
        pltpu.VMEM((NCH, K), jnp.int32),
        pltpu.VMEM((NCH, K), jnp.int32),
        pltpu.VMEM((K, C), jnp.float32),
        pltpu.VMEM((RCH, C), jnp.float32),
        pltpu.VMEM_SHARED((N, C), jnp.float32),
        pltpu.SemaphoreType.DMA,
    ],
)
def _agg(src_hbm, dst_hbm, h_hbm, out_hbm, sidx, didx, rows, zbuf, acc, sem):
    cid = lax.axis_index("c")
    sid = lax.axis_index("s")
    wid = cid * NS + sid

    def zrow(i, carry):
        for c16 in range(C // 16):
            zbuf[i, pl.ds(c16 * 16, 16)] = jnp.zeros((16,), jnp.float32)
        return carry
    lax.fori_loop(0, RCH, zrow, None)
    for t in range(RPT // RCH):
        pltpu.sync_copy(zbuf, acc.at[pl.ds(sid * RPT + t * RCH, RCH)])
    plsc.subcore_barrier()

    pltpu.sync_copy(src_hbm.at[wid], sidx)
    pltpu.sync_copy(dst_hbm.at[wid], didx)

    def chunk(j, carry):
        pltpu.async_copy(h_hbm.at[sidx.at[j]], rows, sem).wait()
        pltpu.sync_copy(rows, acc.at[didx.at[j]], add=True)
        return carry
    lax.fori_loop(0, NCH, chunk, None)

    plsc.subcore_barrier()
    pltpu.sync_copy(acc.at[pl.ds(sid * RPT, RPT)],
                    out_hbm.at[cid, pl.ds(sid * RPT, RPT)])


def _prescale_body(x_ref, degp_ref, h_ref, ns_ref, nd_ref):
    deg = degp_ref[0] + degp_ref[1]
    norm = lax.rsqrt(jnp.maximum(deg, 1.0))
    ns = norm[0]
    nd = norm[1]
    h_ref[...] = x_ref[...] * ns
    ns_ref[...] = ns
    nd_ref[...] = nd


_prescale = pl.pallas_call(
    _prescale_body,
    out_shape=[
        jax.ShapeDtypeStruct((N, C), jnp.float32),
        jax.ShapeDtypeStruct((N, 1), jnp.float32),
        jax.ShapeDtypeStruct((N, 1), jnp.float32),
    ],
)

BM = 2000


def _mm1_body(aggp_ref, nd_ref, ns_ref, w_ref, o_ref):
    a = (aggp_ref[0] + aggp_ref[1]) * nd_ref[...]
    h = jnp.dot(a, w_ref[...], preferred_element_type=jnp.float32)
    o_ref[...] = jnp.maximum(h, 0.0) * ns_ref[...]


_mm1 = pl.pallas_call(
    _mm1_body,
    grid=(N // BM,),
    in_specs=[
        pl.BlockSpec((NC, BM, C), lambda i: (0, i, 0)),
        pl.BlockSpec((BM, 1), lambda i: (i, 0)),
        pl.BlockSpec((BM, 1), lambda i: (i, 0)),
        pl.BlockSpec((C, C), lambda i: (0, 0)),
    ],
    out_specs=pl.BlockSpec((BM, C), lambda i: (i, 0)),
    out_shape=jax.ShapeDtypeStruct((N, C), jnp.float32),
)


def _mm2_body(aggp_ref, nd_ref, w_ref, o_ref):
    a = (aggp_ref[0] + aggp_ref[1]) * nd_ref[...]
    o_ref[...] = jnp.dot(a, w_ref[...], preferred_element_type=jnp.float32)


_mm2 = pl.pallas_call(
    _mm2_body,
    grid=(N // BM,),
    in_specs=[
        pl.BlockSpec((NC, BM, C), lambda i: (0, i, 0)),
        pl.BlockSpec((BM, 1), lambda i: (i, 0)),
        pl.BlockSpec((C, C), lambda i: (0, 0)),
    ],
    out_specs=pl.BlockSpec((BM, C), lambda i: (i, 0)),
    out_shape=jax.ShapeDtypeStruct((N, C), jnp.float32),
)


def kernel(x, edge_index, W1, W2):
    ei = edge_index.astype(jnp.int32)
    src = ei[0].reshape(NW, NCH, K)
    dst = ei[1].reshape(NW, NCH, K)

    degp = _deg(src, dst)
    h, ns, nd = _prescale(x, degp.reshape(NC, 2, N, 1))
    aggp = _agg(src, dst, h)
    h1 = _mm1(aggp, nd, ns, W1)
    aggp2 = _agg(src, dst, h1)
    out = _mm2(aggp2, nd, W2)
    return out


# speedup vs baseline: 7.0616x; 7.0616x over previous
import functools

import jax
import jax.numpy as jnp
from jax import lax
from jax.experimental import pallas as pl
from jax.experimental.pallas import tpu as pltpu
from jax.experimental.pallas import tpu_sc as plsc

N = 10000
E = 320000
C = 128
NC = 2
NS = 16
NW = NC * NS
EPW = E // NW
K = 80
NCH = EPW // K
RPT = N // NS
RCH = 125


def _vsc_mesh():
    return plsc.VectorSubcoreMesh(core_axis_name="c", subcore_axis_name="s")


@functools.partial(
    pl.kernel,
    out_type=jax.ShapeDtypeStruct((NC, 2, N), jnp.float32),
    mesh=_vsc_mesh(),
    scratch_types=[
        pltpu.VMEM((NCH, K), jnp.int32),
        pltpu.VMEM((NCH, K), jnp.int32),
        pltpu.VMEM((K,), jnp.float32),
        pltpu.VMEM((2000,), jnp.float32),
        pltpu.VMEM_SHARED((N,), jnp.float32),
        pltpu.VMEM_SHARED((N,), jnp.float32),
    ],
    compiler_params=pltpu.CompilerParams(use_tc_tiling_on_sc=False),
)
def _deg(src_hbm, dst_hbm, degp_hbm, sidx, didx, ones, zbuf, acc_s, acc_d):
    cid = lax.axis_index("c")
    sid = lax.axis_index("s")
    wid = cid * NS + sid

    @pl.when(sid == 0)
    def _init():
        def zrow(i, carry):
            zbuf[pl.ds(i * 16, 16)] = jnp.zeros((16,), jnp.float32)
            return carry
        lax.fori_loop(0, 2000 // 16, zrow, None)
        for t in range(N // 2000):
            pltpu.sync_copy(zbuf, acc_s.at[pl.ds(t * 2000, 2000)])
            pltpu.sync_copy(zbuf, acc_d.at[pl.ds(t * 2000, 2000)])

    for c16 in range(K // 16):
        ones[pl.ds(c16 * 16, 16)] = jnp.ones((16,), jnp.float32)

    plsc.subcore_barrier()

    pltpu.sync_copy(src_hbm.at[wid], sidx)
    pltpu.sync_copy(dst_hbm.at[wid], didx)

    def chunk(j, carry):
        pltpu.sync_copy(ones, acc_s.at[sidx.at[j]], add=True)
        pltpu.sync_copy(ones, acc_d.at[didx.at[j]], add=True)
        return carry
    lax.fori_loop(0, NCH, chunk, None)

    plsc.subcore_barrier()

    @pl.when(sid == 0)
    def _writeout():
        pltpu.sync_copy(acc_s, degp_hbm.at[cid, 0])
        pltpu.sync_copy(acc_d, degp_hbm.at[cid, 1])


@functools.partial(
    pl.kernel,
    out_type=jax.ShapeDtypeStruct((NC, N, C), jnp.float32),
    mesh=_vsc_mesh(),
    scratch_types=[
        pltpu.VMEM((NCH, K), jnp.int32),
        pltpu.VMEM((NCH, K), jnp.int32),
        pltpu.VMEM((K, C), jnp.float32),
        pltpu.VMEM((RCH, C), jnp.float32),
        pltpu.VMEM_SHARED((N, C), jnp.float32),
        pltpu.SemaphoreType.DMA,
    ],
    compiler_params=pltpu.CompilerParams(use_tc_tiling_on_sc=False),
)
def _agg(src_hbm, dst_hbm, h_hbm, out_hbm, sidx, didx, rows, zbuf, acc, sem):
    cid = lax.axis_index("c")
    sid = lax.axis_index("s")
    wid = cid * NS + sid

    def zrow(i, carry):
        for c16 in range(C // 16):
            zbuf[i, pl.ds(c16 * 16, 16)] = jnp.zeros((16,), jnp.float32)
        return carry
    lax.fori_loop(0, RCH, zrow, None)
    for t in range(RPT // RCH):
        pltpu.sync_copy(zbuf, acc.at[pl.ds(sid * RPT + t * RCH, RCH)])
    plsc.subcore_barrier()

    pltpu.sync_copy(src_hbm.at[wid], sidx)
    pltpu.sync_copy(dst_hbm.at[wid], didx)

    def chunk(j, carry):
        pltpu.async_copy(h_hbm.at[sidx.at[j]], rows, sem).wait()
        pltpu.sync_copy(rows, acc.at[didx.at[j]], add=True)
        return carry
    lax.fori_loop(0, NCH, chunk, None)

    plsc.subcore_barrier()
    pltpu.sync_copy(acc.at[pl.ds(sid * RPT, RPT)],
                    out_hbm.at[cid, pl.ds(sid * RPT, RPT)])


def _prescale_body(x_ref, degp_ref, h_ref, ns_ref, nd_ref):
    deg = degp_ref[0] + degp_ref[1]
    norm = lax.rsqrt(jnp.maximum(deg, 1.0))
    ns = norm[0]
    nd = norm[1]
    h_ref[...] = x_ref[...] * ns
    ns_ref[...] = ns
    nd_ref[...] = nd


_prescale = pl.pallas_call(
    _prescale_body,
    out_shape=[
        jax.ShapeDtypeStruct((N, C), jnp.float32),
        jax.ShapeDtypeStruct((N, 1), jnp.float32),
        jax.ShapeDtypeStruct((N, 1), jnp.float32),
    ],
)

BM = 2000


def _mm1_body(aggp_ref, nd_ref, ns_ref, w_ref, o_ref):
    a = (aggp_ref[0] + aggp_ref[1]) * nd_ref[...]
    h = jnp.dot(a, w_ref[...], preferred_element_type=jnp.float32)
    o_ref[...] = jnp.maximum(h, 0.0) * ns_ref[...]


_mm1 = pl.pallas_call(
    _mm1_body,
    grid=(N // BM,),
    in_specs=[
        pl.BlockSpec((NC, BM, C), lambda i: (0, i, 0)),
        pl.BlockSpec((BM, 1), lambda i: (i, 0)),
        pl.BlockSpec((BM, 1), lambda i: (i, 0)),
        pl.BlockSpec((C, C), lambda i: (0, 0)),
    ],
    out_specs=pl.BlockSpec((BM, C), lambda i: (i, 0)),
    out_shape=jax.ShapeDtypeStruct((N, C), jnp.float32),
)


def _mm2_body(aggp_ref, nd_ref, w_ref, o_ref):
    a = (aggp_ref[0] + aggp_ref[1]) * nd_ref[...]
    o_ref[...] = jnp.dot(a, w_ref[...], preferred_element_type=jnp.float32)


_mm2 = pl.pallas_call(
    _mm2_body,
    grid=(N // BM,),
    in_specs=[
        pl.BlockSpec((NC, BM, C), lambda i: (0, i, 0)),
        pl.BlockSpec((BM, 1), lambda i: (i, 0)),
        pl.BlockSpec((C, C), lambda i: (0, 0)),
    ],
    out_specs=pl.BlockSpec((BM, C), lambda i: (i, 0)),
    out_shape=jax.ShapeDtypeStruct((N, C), jnp.float32),
)


def kernel(x, edge_index, W1, W2):
    ei = edge_index.astype(jnp.int32)
    src = ei[0].reshape(NW, NCH, K)
    dst = ei[1].reshape(NW, NCH, K)

    degp = _deg(src, dst)
    h, ns, nd = _prescale(x, degp.reshape(NC, 2, N, 1))
    aggp = _agg(src, dst, h)
    h1 = _mm1(aggp, nd, ns, W1)
    aggp2 = _agg(src, dst, h1)
    out = _mm2(aggp2, nd, W2)
    return out


# speedup vs baseline: 11.2198x; 1.5888x over previous
import functools

import jax
import jax.numpy as jnp
from jax import lax
from jax.experimental import pallas as pl
from jax.experimental.pallas import tpu as pltpu
from jax.experimental.pallas import tpu_sc as plsc

N = 10000
E = 320000
C = 128
NC = 2
NS = 16
NW = NC * NS
EPW = E // NW
KD = 80
NCHD = EPW // KD
K = 125
NCH = EPW // K
NPH = 2
HCH = NCH // NPH
NBUF = 2
RPT = N // NS
RCH = 125


def _vsc_mesh():
    return plsc.VectorSubcoreMesh(core_axis_name="c", subcore_axis_name="s")


@functools.partial(
    pl.kernel,
    out_type=jax.ShapeDtypeStruct((NC, 2, N), jnp.float32),
    mesh=_vsc_mesh(),
    scratch_types=[
        pltpu.VMEM((NCHD, KD), jnp.int32),
        pltpu.VMEM((NCHD, KD), jnp.int32),
        pltpu.VMEM((KD,), jnp.float32),
        pltpu.VMEM((2000,), jnp.float32),
        pltpu.VMEM_SHARED((N,), jnp.float32),
        pltpu.VMEM_SHARED((N,), jnp.float32),
    ],
    compiler_params=pltpu.CompilerParams(use_tc_tiling_on_sc=False),
)
def _deg(src_hbm, dst_hbm, degp_hbm, sidx, didx, ones, zbuf, acc_s, acc_d):
    cid = lax.axis_index("c")
    sid = lax.axis_index("s")
    wid = cid * NS + sid

    @pl.when(sid == 0)
    def _init():
        def zrow(i, carry):
            zbuf[pl.ds(i * 16, 16)] = jnp.zeros((16,), jnp.float32)
            return carry
        lax.fori_loop(0, 2000 // 16, zrow, None)
        for t in range(N // 2000):
            pltpu.sync_copy(zbuf, acc_s.at[pl.ds(t * 2000, 2000)])
            pltpu.sync_copy(zbuf, acc_d.at[pl.ds(t * 2000, 2000)])

    for c16 in range(KD // 16):
        ones[pl.ds(c16 * 16, 16)] = jnp.ones((16,), jnp.float32)

    plsc.subcore_barrier()

    pltpu.sync_copy(src_hbm.at[wid], sidx)
    pltpu.sync_copy(dst_hbm.at[wid], didx)

    def chunk(j, carry):
        pltpu.sync_copy(ones, acc_s.at[sidx.at[j]], add=True)
        pltpu.sync_copy(ones, acc_d.at[didx.at[j]], add=True)
        return carry
    lax.fori_loop(0, NCHD, chunk, None)

    plsc.subcore_barrier()

    @pl.when(sid == 0)
    def _writeout():
        pltpu.sync_copy(acc_s, degp_hbm.at[cid, 0])
        pltpu.sync_copy(acc_d, degp_hbm.at[cid, 1])


@functools.partial(
    pl.kernel,
    out_type=jax.ShapeDtypeStruct((NC, N, C), jnp.float32),
    mesh=_vsc_mesh(),
    scratch_types=[
        pltpu.VMEM((HCH, K), jnp.int32),
        pltpu.VMEM((HCH, K), jnp.int32),
        [pltpu.VMEM((K, C), jnp.float32) for _ in range(NBUF)],
        [pltpu.SemaphoreType.DMA for _ in range(NBUF)],
        pltpu.VMEM_SHARED((N, C), jnp.float32),
    ],
    compiler_params=pltpu.CompilerParams(use_tc_tiling_on_sc=False),
)
def _agg(src_hbm, dst_hbm, h_hbm, out_hbm, sidx, didx, rows, sems, acc):
    cid = lax.axis_index("c")
    sid = lax.axis_index("s")
    wid = cid * NS + sid

    def zrow(i, carry):
        for c16 in range(C // 16):
            rows[0][i, pl.ds(c16 * 16, 16)] = jnp.zeros((16,), jnp.float32)
        return carry
    lax.fori_loop(0, K, zrow, None)
    for t in range(RPT // K):
        pltpu.sync_copy(rows[0], acc.at[pl.ds(sid * RPT + t * K, K)])
    plsc.subcore_barrier()

    for p in range(NPH):
        pltpu.sync_copy(src_hbm.at[wid, p], sidx)
        pltpu.sync_copy(dst_hbm.at[wid, p], didx)

        for b in range(NBUF):
            pltpu.async_copy(h_hbm.at[sidx.at[b]], rows[b], sems[b])

        def outer(o, carry):
            base = o * NBUF
            for b in range(NBUF):
                j = base + b
                pltpu.make_async_copy(h_hbm.at[pl.ds(0, K)], rows[b],
                                      sems[b]).wait()
                pltpu.sync_copy(rows[b], acc.at[didx.at[j]], add=True)
                pltpu.async_copy(h_hbm.at[sidx.at[j + NBUF]], rows[b],
                                 sems[b])
            return carry
        lax.fori_loop(0, HCH // NBUF - 1, outer, None)

        base = HCH - NBUF
        for b in range(NBUF):
            pltpu.make_async_copy(h_hbm.at[pl.ds(0, K)], rows[b],
                                  sems[b]).wait()
            pltpu.sync_copy(rows[b], acc.at[didx.at[base + b]], add=True)

    plsc.subcore_barrier()
    pltpu.sync_copy(acc.at[pl.ds(sid * RPT, RPT)],
                    out_hbm.at[cid, pl.ds(sid * RPT, RPT)])


def _prescale_body(x_ref, degp_ref, h_ref, ns_ref, nd_ref):
    deg = degp_ref[0] + degp_ref[1]
    norm = lax.rsqrt(jnp.maximum(deg, 1.0))
    ns = norm[0]
    nd = norm[1]
    h_ref[...] = x_ref[...] * ns
    ns_ref[...] = ns
    nd_ref[...] = nd


_prescale = pl.pallas_call(
    _prescale_body,
    out_shape=[
        jax.ShapeDtypeStruct((N, C), jnp.float32),
        jax.ShapeDtypeStruct((N, 1), jnp.float32),
        jax.ShapeDtypeStruct((N, 1), jnp.float32),
    ],
)

BM = 2000


def _mm1_body(aggp_ref, nd_ref, ns_ref, w_ref, o_ref):
    a = (aggp_ref[0] + aggp_ref[1]) * nd_ref[...]
    h = jnp.dot(a, w_ref[...], preferred_element_type=jnp.float32)
    o_ref[...] = jnp.maximum(h, 0.0) * ns_ref[...]


_mm1 = pl.pallas_call(
    _mm1_body,
    grid=(N // BM,),
    in_specs=[
        pl.BlockSpec((NC, BM, C), lambda i: (0, i, 0)),
        pl.BlockSpec((BM, 1), lambda i: (i, 0)),
        pl.BlockSpec((BM, 1), lambda i: (i, 0)),
        pl.BlockSpec((C, C), lambda i: (0, 0)),
    ],
    out_specs=pl.BlockSpec((BM, C), lambda i: (i, 0)),
    out_shape=jax.ShapeDtypeStruct((N, C), jnp.float32),
)


def _mm2_body(aggp_ref, nd_ref, w_ref, o_ref):
    a = (aggp_ref[0] + aggp_ref[1]) * nd_ref[...]
    o_ref[...] = jnp.dot(a, w_ref[...], preferred_element_type=jnp.float32)


_mm2 = pl.pallas_call(
    _mm2_body,
    grid=(N // BM,),
    in_specs=[
        pl.BlockSpec((NC, BM, C), lambda i: (0, i, 0)),
        pl.BlockSpec((BM, 1), lambda i: (i, 0)),
        pl.BlockSpec((C, C), lambda i: (0, 0)),
    ],
    out_specs=pl.BlockSpec((BM, C), lambda i: (i, 0)),
    out_shape=jax.ShapeDtypeStruct((N, C), jnp.float32),
)


def kernel(x, edge_index, W1, W2):
    ei = edge_index.astype(jnp.int32)
    src_d = ei[0].reshape(NW, NCHD, KD)
    dst_d = ei[1].reshape(NW, NCHD, KD)
    src_a = ei[0].reshape(NW, NPH, HCH, K)
    dst_a = ei[1].reshape(NW, NPH, HCH, K)

    degp = _deg(src_d, dst_d)
    h, ns, nd = _prescale(x, degp.reshape(NC, 2, N, 1))
    aggp = _agg(src_a, dst_a, h)
    h1 = _mm1(aggp, nd, ns, W1)
    aggp2 = _agg(src_a, dst_a, h1)
    out = _mm2(aggp2, nd, W2)
    return out


# speedup vs baseline: 12.0266x; 1.0719x over previous
import functools

import jax
import jax.numpy as jnp
from jax import lax
from jax.experimental import pallas as pl
from jax.experimental.pallas import tpu as pltpu
from jax.experimental.pallas import tpu_sc as plsc

N = 10000
E = 320000
C = 128
NC = 2
NS = 16
NW = NC * NS
EPW = E // NW
KD = 80
NCHD = EPW // KD
K = 50
NCH = EPW // K
NPH = 4
HCH = NCH // NPH
NBUF = 5
RPT = N // NS
RCH = 125


def _vsc_mesh():
    return plsc.VectorSubcoreMesh(core_axis_name="c", subcore_axis_name="s")


@functools.partial(
    pl.kernel,
    out_type=jax.ShapeDtypeStruct((NC, 2, N), jnp.float32),
    mesh=_vsc_mesh(),
    scratch_types=[
        pltpu.VMEM((NCHD, KD), jnp.int32),
        pltpu.VMEM((NCHD, KD), jnp.int32),
        pltpu.VMEM((KD,), jnp.float32),
        pltpu.VMEM((2000,), jnp.float32),
        pltpu.VMEM_SHARED((N,), jnp.float32),
        pltpu.VMEM_SHARED((N,), jnp.float32),
        pltpu.SemaphoreType.DMA,
    ],
    compiler_params=pltpu.CompilerParams(use_tc_tiling_on_sc=False),
)
def _deg(src_hbm, dst_hbm, degp_hbm, sidx, didx, ones, zbuf, acc_s, acc_d,
         dsem):
    cid = lax.axis_index("c")
    sid = lax.axis_index("s")
    wid = cid * NS + sid

    @pl.when(sid == 0)
    def _init():
        def zrow(i, carry):
            zbuf[pl.ds(i * 16, 16)] = jnp.zeros((16,), jnp.float32)
            return carry
        lax.fori_loop(0, 2000 // 16, zrow, None)
        for t in range(N // 2000):
            pltpu.sync_copy(zbuf, acc_s.at[pl.ds(t * 2000, 2000)])
            pltpu.sync_copy(zbuf, acc_d.at[pl.ds(t * 2000, 2000)])

    for c16 in range(KD // 16):
        ones[pl.ds(c16 * 16, 16)] = jnp.ones((16,), jnp.float32)

    plsc.subcore_barrier()

    pltpu.sync_copy(src_hbm.at[wid], sidx)
    pltpu.sync_copy(dst_hbm.at[wid], didx)

    DB = 5

    def _fire(base):
        for b in range(DB):
            pltpu.async_copy(ones, acc_s.at[sidx.at[base + b]], dsem,
                             add=True)
            pltpu.async_copy(ones, acc_d.at[didx.at[base + b]], dsem,
                             add=True)

    def _drain():
        for _ in range(2 * DB):
            pltpu.make_async_copy(ones, acc_s.at[sidx.at[0]], dsem).wait()

    _fire(0)

    def chunk(o, carry):
        _fire((o + 1) * DB)
        _drain()
        return carry
    lax.fori_loop(0, NCHD // DB - 1, chunk, None)
    _drain()

    plsc.subcore_barrier()

    @pl.when(sid == 0)
    def _writeout():
        pltpu.sync_copy(acc_s, degp_hbm.at[cid, 0])
        pltpu.sync_copy(acc_d, degp_hbm.at[cid, 1])


@functools.partial(
    pl.kernel,
    out_type=jax.ShapeDtypeStruct((NC, N, C), jnp.float32),
    mesh=_vsc_mesh(),
    scratch_types=[
        pltpu.VMEM((HCH, K), jnp.int32),
        pltpu.VMEM((HCH, K), jnp.int32),
        [pltpu.VMEM((K, C), jnp.float32) for _ in range(NBUF)],
        [pltpu.SemaphoreType.DMA for _ in range(NBUF)],
        pltpu.VMEM_SHARED((N, C), jnp.float32),
    ],
    compiler_params=pltpu.CompilerParams(use_tc_tiling_on_sc=False),
)
def _agg(src_hbm, dst_hbm, h_hbm, out_hbm, sidx, didx, rows, sems, acc):
    cid = lax.axis_index("c")
    sid = lax.axis_index("s")
    wid = cid * NS + sid

    def zrow(i, carry):
        for c16 in range(C // 16):
            rows[0][i, pl.ds(c16 * 16, 16)] = jnp.zeros((16,), jnp.float32)
        return carry
    lax.fori_loop(0, K, zrow, None)
    for t in range(RPT // K):
        pltpu.sync_copy(rows[0], acc.at[pl.ds(sid * RPT + t * K, K)])
    _REM = RPT - (RPT // K) * K
    if _REM:
        pltpu.sync_copy(rows[0].at[pl.ds(0, _REM)],
                        acc.at[pl.ds(sid * RPT + (RPT // K) * K, _REM)])
    plsc.subcore_barrier()

    for p in range(NPH):
        pltpu.sync_copy(src_hbm.at[wid, p], sidx)
        pltpu.sync_copy(dst_hbm.at[wid, p], didx)

        for b in range(NBUF):
            pltpu.async_copy(h_hbm.at[sidx.at[b]], rows[b], sems[b])

        def outer(o, carry):
            base = o * NBUF
            for b in range(NBUF):
                j = base + b
                pltpu.make_async_copy(h_hbm.at[pl.ds(0, K)], rows[b],
                                      sems[b]).wait()
                pltpu.sync_copy(rows[b], acc.at[didx.at[j]], add=True)
                pltpu.async_copy(h_hbm.at[sidx.at[j + NBUF]], rows[b],
                                 sems[b])
            return carry
        lax.fori_loop(0, HCH // NBUF - 1, outer, None)

        base = HCH - NBUF
        for b in range(NBUF):
            pltpu.make_async_copy(h_hbm.at[pl.ds(0, K)], rows[b],
                                  sems[b]).wait()
            pltpu.sync_copy(rows[b], acc.at[didx.at[base + b]], add=True)

    plsc.subcore_barrier()
    pltpu.sync_copy(acc.at[pl.ds(sid * RPT, RPT)],
                    out_hbm.at[cid, pl.ds(sid * RPT, RPT)])


def _prescale_body(x_ref, degp_ref, h_ref, ns_ref, nd_ref):
    deg = degp_ref[0] + degp_ref[1]
    norm = lax.rsqrt(jnp.maximum(deg, 1.0))
    ns = norm[0]
    nd = norm[1]
    h_ref[...] = x_ref[...] * ns
    ns_ref[...] = ns
    nd_ref[...] = nd


_prescale = pl.pallas_call(
    _prescale_body,
    out_shape=[
        jax.ShapeDtypeStruct((N, C), jnp.float32),
        jax.ShapeDtypeStruct((N, 1), jnp.float32),
        jax.ShapeDtypeStruct((N, 1), jnp.float32),
    ],
)

BM = 2000


def _mm1_body(aggp_ref, nd_ref, ns_ref, w_ref, o_ref):
    a = (aggp_ref[0] + aggp_ref[1]) * nd_ref[...]
    h = jnp.dot(a, w_ref[...], preferred_element_type=jnp.float32)
    o_ref[...] = jnp.maximum(h, 0.0) * ns_ref[...]


_mm1 = pl.pallas_call(
    _mm1_body,
    grid=(N // BM,),
    in_specs=[
        pl.BlockSpec((NC, BM, C), lambda i: (0, i, 0)),
        pl.BlockSpec((BM, 1), lambda i: (i, 0)),
        pl.BlockSpec((BM, 1), lambda i: (i, 0)),
        pl.BlockSpec((C, C), lambda i: (0, 0)),
    ],
    out_specs=pl.BlockSpec((BM, C), lambda i: (i, 0)),
    out_shape=jax.ShapeDtypeStruct((N, C), jnp.float32),
)


def _mm2_body(aggp_ref, nd_ref, w_ref, o_ref):
    a = (aggp_ref[0] + aggp_ref[1]) * nd_ref[...]
    o_ref[...] = jnp.dot(a, w_ref[...], preferred_element_type=jnp.float32)


_mm2 = pl.pallas_call(
    _mm2_body,
    grid=(N // BM,),
    in_specs=[
        pl.BlockSpec((NC, BM, C), lambda i: (0, i, 0)),
        pl.BlockSpec((BM, 1), lambda i: (i, 0)),
        pl.BlockSpec((C, C), lambda i: (0, 0)),
    ],
    out_specs=pl.BlockSpec((BM, C), lambda i: (i, 0)),
    out_shape=jax.ShapeDtypeStruct((N, C), jnp.float32),
)


def kernel(x, edge_index, W1, W2):
    ei = edge_index.astype(jnp.int32)
    src_d = ei[0].reshape(NW, NCHD, KD)
    dst_d = ei[1].reshape(NW, NCHD, KD)
    src_a = ei[0].reshape(NW, NPH, HCH, K)
    dst_a = ei[1].reshape(NW, NPH, HCH, K)

    degp = _deg(src_d, dst_d)
    h, ns, nd = _prescale(x, degp.reshape(NC, 2, N, 1))
    aggp = _agg(src_a, dst_a, h)
    h1 = _mm1(aggp, nd, ns, W1)
    aggp2 = _agg(src_a, dst_a, h1)
    out = _mm2(aggp2, nd, W2)
    return out


# speedup vs baseline: 12.7156x; 1.0573x over previous
import functools

import jax
import jax.numpy as jnp
from jax import lax
from jax.experimental import pallas as pl
from jax.experimental.pallas import tpu as pltpu
from jax.experimental.pallas import tpu_sc as plsc

N = 10000
E = 320000
C = 128
NC = 2
NS = 16
NW = NC * NS
EPW = E // NW
K = 80
NCH = EPW // K
NBUF = 3
RPT = N // NS


def _vsc_mesh():
    return plsc.VectorSubcoreMesh(core_axis_name="c", subcore_axis_name="s")


@functools.partial(
    pl.kernel,
    out_type=jax.ShapeDtypeStruct((NC, 2, N), jnp.float32),
    mesh=_vsc_mesh(),
    scratch_types=[
        pltpu.VMEM((EPW,), jnp.int32),
        pltpu.VMEM((EPW,), jnp.int32),
        pltpu.VMEM((K,), jnp.float32),
        pltpu.VMEM((2000,), jnp.float32),
        pltpu.VMEM_SHARED((N,), jnp.float32),
        pltpu.VMEM_SHARED((N,), jnp.float32),
        pltpu.SemaphoreType.DMA,
    ],
    compiler_params=pltpu.CompilerParams(use_tc_tiling_on_sc=False),
)
def _deg(src_hbm, dst_hbm, degp_hbm, sidx, didx, ones, zbuf, acc_s, acc_d,
         dsem):
    cid = lax.axis_index("c")
    sid = lax.axis_index("s")
    wid = cid * NS + sid

    @pl.when(sid == 0)
    def _init():
        def zrow(i, carry):
            zbuf[pl.ds(i * 16, 16)] = jnp.zeros((16,), jnp.float32)
            return carry
        lax.fori_loop(0, 2000 // 16, zrow, None)
        for t in range(N // 2000):
            pltpu.sync_copy(zbuf, acc_s.at[pl.ds(t * 2000, 2000)])
            pltpu.sync_copy(zbuf, acc_d.at[pl.ds(t * 2000, 2000)])

    for c16 in range(K // 16):
        ones[pl.ds(c16 * 16, 16)] = jnp.ones((16,), jnp.float32)

    plsc.subcore_barrier()

    base = pl.multiple_of(wid * EPW, 8)
    pltpu.sync_copy(src_hbm.at[pl.ds(base, EPW)], sidx)
    pltpu.sync_copy(dst_hbm.at[pl.ds(base, EPW)], didx)

    DB = 5

    def _cidx(ref, j):
        return ref.at[pl.ds(pl.multiple_of(j * K, 8), K)]

    def _fire(b0):
        for b in range(DB):
            pltpu.async_copy(ones, acc_s.at[_cidx(sidx, b0 + b)], dsem,
                             add=True)
            pltpu.async_copy(ones, acc_d.at[_cidx(didx, b0 + b)], dsem,
                             add=True)

    def _drain():
        for _ in range(2 * DB):
            pltpu.make_async_copy(ones, acc_s.at[_cidx(sidx, 0)],
                                  dsem).wait()

    _fire(0)

    def chunk(o, carry):
        _fire((o + 1) * DB)
        _drain()
        return carry
    lax.fori_loop(0, NCH // DB - 1, chunk, None)
    _drain()

    plsc.subcore_barrier()

    @pl.when(sid == 0)
    def _writeout():
        pltpu.sync_copy(acc_s, degp_hbm.at[cid, 0])
        pltpu.sync_copy(acc_d, degp_hbm.at[cid, 1])


@functools.partial(
    pl.kernel,
    out_type=jax.ShapeDtypeStruct((NC, N, C), jnp.float32),
    mesh=_vsc_mesh(),
    scratch_types=[
        pltpu.VMEM((EPW,), jnp.int32),
        pltpu.VMEM((EPW,), jnp.int32),
        [pltpu.VMEM((K, C), jnp.float32) for _ in range(NBUF)],
        [pltpu.SemaphoreType.DMA for _ in range(NBUF)],
        pltpu.VMEM_SHARED((N, C), jnp.float32),
    ],
    compiler_params=pltpu.CompilerParams(use_tc_tiling_on_sc=False),
)
def _agg(src_hbm, dst_hbm, h_hbm, out_hbm, sidx, didx, rows, sems, acc):
    cid = lax.axis_index("c")
    sid = lax.axis_index("s")
    wid = cid * NS + sid

    def zrow(i, carry):
        for c16 in range(C // 16):
            rows[0][i, pl.ds(c16 * 16, 16)] = jnp.zeros((16,), jnp.float32)
        return carry
    lax.fori_loop(0, K, zrow, None)
    for t in range(RPT // K):
        pltpu.sync_copy(rows[0], acc.at[pl.ds(sid * RPT + t * K, K)])
    _REM = RPT - (RPT // K) * K
    if _REM:
        pltpu.sync_copy(rows[0].at[pl.ds(0, _REM)],
                        acc.at[pl.ds(sid * RPT + (RPT // K) * K, _REM)])
    plsc.subcore_barrier()

    base = pl.multiple_of(wid * EPW, 8)
    pltpu.sync_copy(src_hbm.at[pl.ds(base, EPW)], sidx)
    pltpu.sync_copy(dst_hbm.at[pl.ds(base, EPW)], didx)

    def _cidx(ref, j):
        return ref.at[pl.ds(pl.multiple_of(j * K, 8), K)]

    def _gather(j, b):
        pltpu.async_copy(h_hbm.at[_cidx(sidx, j)], rows[b], sems[b])

    def _scat(j, b):
        pltpu.make_async_copy(h_hbm.at[pl.ds(0, K)], rows[b], sems[b]).wait()
        pltpu.sync_copy(rows[b], acc.at[_cidx(didx, j)], add=True)

    OUT = NCH // NBUF - 1
    TAIL = NCH - NBUF * OUT

    for b in range(NBUF):
        _gather(b, b)

    def outer(o, carry):
        jb = o * NBUF
        for b in range(NBUF):
            _scat(jb + b, b)
            _gather(jb + b + NBUF, b)
        return carry
    lax.fori_loop(0, OUT, outer, None)

    jb = OUT * NBUF
    for b in range(NBUF):
        _scat(jb + b, b)
    for t in range(TAIL - NBUF):
        j = jb + NBUF + t
        _gather(j, t)
        _scat(j, t)

    plsc.subcore_barrier()
    pltpu.sync_copy(acc.at[pl.ds(sid * RPT, RPT)],
                    out_hbm.at[cid, pl.ds(sid * RPT, RPT)])


BM = 2000


def _prescale_body(x_ref, degp_ref, h_ref, ns_ref, nd_ref):
    deg = degp_ref[0] + degp_ref[1]
    norm = lax.rsqrt(jnp.maximum(deg, 1.0))
    ns = norm[0]
    nd = norm[1]
    h_ref[...] = x_ref[...] * ns
    ns_ref[...] = ns
    nd_ref[...] = nd


_prescale = pl.pallas_call(
    _prescale_body,
    grid=(N // BM,),
    in_specs=[
        pl.BlockSpec((BM, C), lambda i: (i, 0)),
        pl.BlockSpec((NC, 2, BM, 1), lambda i: (0, 0, i, 0)),
    ],
    out_specs=[
        pl.BlockSpec((BM, C), lambda i: (i, 0)),
        pl.BlockSpec((BM, 1), lambda i: (i, 0)),
        pl.BlockSpec((BM, 1), lambda i: (i, 0)),
    ],
    out_shape=[
        jax.ShapeDtypeStruct((N, C), jnp.float32),
        jax.ShapeDtypeStruct((N, 1), jnp.float32),
        jax.ShapeDtypeStruct((N, 1), jnp.float32),
    ],
)


def _mm1_body(aggp_ref, nd_ref, ns_ref, w_ref, o_ref):
    a = (aggp_ref[0] + aggp_ref[1]) * nd_ref[...]
    h = jnp.dot(a, w_ref[...], preferred_element_type=jnp.float32)
    o_ref[...] = jnp.maximum(h, 0.0) * ns_ref[...]


_mm1 = pl.pallas_call(
    _mm1_body,
    grid=(N // BM,),
    in_specs=[
        pl.BlockSpec((NC, BM, C), lambda i: (0, i, 0)),
        pl.BlockSpec((BM, 1), lambda i: (i, 0)),
        pl.BlockSpec((BM, 1), lambda i: (i, 0)),
        pl.BlockSpec((C, C), lambda i: (0, 0)),
    ],
    out_specs=pl.BlockSpec((BM, C), lambda i: (i, 0)),
    out_shape=jax.ShapeDtypeStruct((N, C), jnp.float32),
)


def _mm2_body(aggp_ref, nd_ref, w_ref, o_ref):
    a = (aggp_ref[0] + aggp_ref[1]) * nd_ref[...]
    o_ref[...] = jnp.dot(a, w_ref[...], preferred_element_type=jnp.float32)


_mm2 = pl.pallas_call(
    _mm2_body,
    grid=(N // BM,),
    in_specs=[
        pl.BlockSpec((NC, BM, C), lambda i: (0, i, 0)),
        pl.BlockSpec((BM, 1), lambda i: (i, 0)),
        pl.BlockSpec((C, C), lambda i: (0, 0)),
    ],
    out_specs=pl.BlockSpec((BM, C), lambda i: (i, 0)),
    out_shape=jax.ShapeDtypeStruct((N, C), jnp.float32),
)


def kernel(x, edge_index, W1, W2):
    ei = edge_index.astype(jnp.int32)
    src = ei[0]
    dst = ei[1]

    degp = _deg(src, dst)
    h, ns, nd = _prescale(x, degp.reshape(NC, 2, N, 1))
    aggp = _agg(src, dst, h)
    h1 = _mm1(aggp, nd, ns, W1)
    aggp2 = _agg(src, dst, h1)
    out = _mm2(aggp2, nd, W2)
    return out


# speedup vs baseline: 14.0727x; 1.1067x over previous
import functools

import jax
import jax.numpy as jnp
from jax import lax
from jax.experimental import pallas as pl
from jax.experimental.pallas import tpu as pltpu
from jax.experimental.pallas import tpu_sc as plsc

N = 10000
E = 320000
C = 128
NC = 2
NS = 16
NW = NC * NS
EPW = E // NW
K = 80
NCH = EPW // K
NBUF = 3
RPT = N // NS


def _vsc_mesh():
    return plsc.VectorSubcoreMesh(core_axis_name="c", subcore_axis_name="s")


@functools.partial(
    pl.kernel,
    out_type=jax.ShapeDtypeStruct((NC, 2, N), jnp.float32),
    mesh=_vsc_mesh(),
    scratch_types=[
        pltpu.VMEM((EPW,), jnp.int32),
        pltpu.VMEM((EPW,), jnp.int32),
        pltpu.VMEM((K,), jnp.float32),
        pltpu.VMEM((2000,), jnp.float32),
        pltpu.VMEM_SHARED((N,), jnp.float32),
        pltpu.VMEM_SHARED((N,), jnp.float32),
        pltpu.SemaphoreType.DMA,
    ],
    compiler_params=pltpu.CompilerParams(use_tc_tiling_on_sc=False),
)
def _deg(src_hbm, dst_hbm, degp_hbm, sidx, didx, ones, zbuf, acc_s, acc_d,
         dsem):
    cid = lax.axis_index("c")
    sid = lax.axis_index("s")
    wid = cid * NS + sid

    @pl.when(sid == 0)
    def _init():
        def zrow(i, carry):
            zbuf[pl.ds(i * 16, 16)] = jnp.zeros((16,), jnp.float32)
            return carry
        lax.fori_loop(0, 2000 // 16, zrow, None)
        for t in range(N // 2000):
            pltpu.sync_copy(zbuf, acc_s.at[pl.ds(t * 2000, 2000)])
            pltpu.sync_copy(zbuf, acc_d.at[pl.ds(t * 2000, 2000)])

    for c16 in range(K // 16):
        ones[pl.ds(c16 * 16, 16)] = jnp.ones((16,), jnp.float32)

    plsc.subcore_barrier()

    base = pl.multiple_of(wid * EPW, 8)
    pltpu.sync_copy(src_hbm.at[pl.ds(base, EPW)], sidx)
    pltpu.sync_copy(dst_hbm.at[pl.ds(base, EPW)], didx)

    DB = 5

    def _cidx(ref, j):
        return ref.at[pl.ds(pl.multiple_of(j * K, 8), K)]

    def _fire(b0):
        for b in range(DB):
            pltpu.async_copy(ones, acc_s.at[_cidx(sidx, b0 + b)], dsem,
                             add=True)
            pltpu.async_copy(ones, acc_d.at[_cidx(didx, b0 + b)], dsem,
                             add=True)

    def _drain():
        for _ in range(2 * DB):
            pltpu.make_async_copy(ones, acc_s.at[_cidx(sidx, 0)],
                                  dsem).wait()

    _fire(0)

    def chunk(o, carry):
        _fire((o + 1) * DB)
        _drain()
        return carry
    lax.fori_loop(0, NCH // DB - 1, chunk, None)
    _drain()

    plsc.subcore_barrier()

    @pl.when(sid == 0)
    def _writeout():
        pltpu.sync_copy(acc_s, degp_hbm.at[cid, 0])
        pltpu.sync_copy(acc_d, degp_hbm.at[cid, 1])


@functools.partial(
    pl.kernel,
    out_type=jax.ShapeDtypeStruct((NC, N, C), jnp.float32),
    mesh=_vsc_mesh(),
    scratch_types=[
        pltpu.VMEM((EPW,), jnp.int32),
        pltpu.VMEM((EPW,), jnp.int32),
        [pltpu.VMEM((K, C), jnp.float32) for _ in range(NBUF)],
        [pltpu.SemaphoreType.DMA for _ in range(NBUF)],
        pltpu.VMEM_SHARED((N, C), jnp.float32),
    ],
    compiler_params=pltpu.CompilerParams(use_tc_tiling_on_sc=False),
)
def _agg(src_hbm, dst_hbm, h_hbm, out_hbm, sidx, didx, rows, sems, acc):
    cid = lax.axis_index("c")
    sid = lax.axis_index("s")
    wid = cid * NS + sid

    def zrow(i, carry):
        for c16 in range(C // 16):
            rows[0][i, pl.ds(c16 * 16, 16)] = jnp.zeros((16,), jnp.float32)
        return carry
    lax.fori_loop(0, K, zrow, None)
    for t in range(RPT // K):
        pltpu.sync_copy(rows[0], acc.at[pl.ds(sid * RPT + t * K, K)])
    _REM = RPT - (RPT // K) * K
    if _REM:
        pltpu.sync_copy(rows[0].at[pl.ds(0, _REM)],
                        acc.at[pl.ds(sid * RPT + (RPT // K) * K, _REM)])
    plsc.subcore_barrier()

    base = pl.multiple_of(wid * EPW, 8)
    pltpu.sync_copy(src_hbm.at[pl.ds(base, EPW)], sidx)
    pltpu.sync_copy(dst_hbm.at[pl.ds(base, EPW)], didx)

    def _cidx(ref, j):
        return ref.at[pl.ds(pl.multiple_of(j * K, 8), K)]

    def _gather(j, b):
        pltpu.async_copy(h_hbm.at[_cidx(sidx, j)], rows[b], sems[b])

    def _scat(j, b):
        pltpu.make_async_copy(h_hbm.at[pl.ds(0, K)], rows[b], sems[b]).wait()
        pltpu.sync_copy(rows[b], acc.at[_cidx(didx, j)], add=True)

    OUT = NCH // NBUF - 1
    TAIL = NCH - NBUF * OUT

    for b in range(NBUF):
        _gather(b, b)

    def outer(o, carry):
        jb = o * NBUF
        for b in range(NBUF):
            _scat(jb + b, b)
            _gather(jb + b + NBUF, b)
        return carry
    lax.fori_loop(0, OUT, outer, None)

    jb = OUT * NBUF
    for b in range(NBUF):
        _scat(jb + b, b)
    for t in range(TAIL - NBUF):
        j = jb + NBUF + t
        _gather(j, t)
        _scat(j, t)

    plsc.subcore_barrier()
    pltpu.sync_copy(acc.at[pl.ds(sid * RPT, RPT)],
                    out_hbm.at[cid, pl.ds(sid * RPT, RPT)])


BM = 2048


def _norm_cols(degp_ref):
    deg = degp_ref[0] + degp_ref[1]
    nrm = lax.rsqrt(jnp.maximum(deg, 1.0))
    nt = jnp.transpose(nrm, (1, 0))
    return nt[:, 0:1], nt[:, 1:2]


def _prescale_body(x_ref, degp_ref, h_ref):
    ns, _ = _norm_cols(degp_ref)
    h_ref[...] = x_ref[...] * ns


_prescale = pl.pallas_call(
    _prescale_body,
    grid=((N + BM - 1) // BM,),
    in_specs=[
        pl.BlockSpec((BM, C), lambda i: (i, 0)),
        pl.BlockSpec((NC, 2, BM), lambda i: (0, 0, i)),
    ],
    out_specs=pl.BlockSpec((BM, C), lambda i: (i, 0)),
    out_shape=jax.ShapeDtypeStruct((N, C), jnp.float32),
)


def _mm1_body(aggp_ref, degp_ref, w_ref, o_ref):
    ns, nd = _norm_cols(degp_ref)
    a = (aggp_ref[0] + aggp_ref[1]) * nd
    h = jnp.dot(a, w_ref[...], preferred_element_type=jnp.float32)
    o_ref[...] = jnp.maximum(h, 0.0) * ns


_mm1 = pl.pallas_call(
    _mm1_body,
    grid=((N + BM - 1) // BM,),
    in_specs=[
        pl.BlockSpec((NC, BM, C), lambda i: (0, i, 0)),
        pl.BlockSpec((NC, 2, BM), lambda i: (0, 0, i)),
        pl.BlockSpec((C, C), lambda i: (0, 0)),
    ],
    out_specs=pl.BlockSpec((BM, C), lambda i: (i, 0)),
    out_shape=jax.ShapeDtypeStruct((N, C), jnp.float32),
)


def _mm2_body(aggp_ref, degp_ref, w_ref, o_ref):
    _, nd = _norm_cols(degp_ref)
    a = (aggp_ref[0] + aggp_ref[1]) * nd
    o_ref[...] = jnp.dot(a, w_ref[...], preferred_element_type=jnp.float32)


_mm2 = pl.pallas_call(
    _mm2_body,
    grid=((N + BM - 1) // BM,),
    in_specs=[
        pl.BlockSpec((NC, BM, C), lambda i: (0, i, 0)),
        pl.BlockSpec((NC, 2, BM), lambda i: (0, 0, i)),
        pl.BlockSpec((C, C), lambda i: (0, 0)),
    ],
    out_specs=pl.BlockSpec((BM, C), lambda i: (i, 0)),
    out_shape=jax.ShapeDtypeStruct((N, C), jnp.float32),
)


def kernel(x, edge_index, W1, W2):
    ei = edge_index.astype(jnp.int32)
    src = ei[0]
    dst = ei[1]

    degp = _deg(src, dst)
    h = _prescale(x, degp)
    aggp = _agg(src, dst, h)
    h1 = _mm1(aggp, degp, W1)
    aggp2 = _agg(src, dst, h1)
    out = _mm2(aggp2, degp, W2)
    return out
